# Initial kernel scaffold; baseline (speedup 1.0000x reference)
import jax
import jax.numpy as jnp
from jax.experimental import pallas as pl


def kernel(T2_d8, T1_d8, T2_d16, T1_d16, batch, params):
    raise NotImplementedError("write your pallas kernel here")


# speedup vs baseline: 4.7741x; 4.7741x over previous
import jax
import jax.numpy as jnp
from jax import lax
from jax.experimental import pallas as pl
import functools

HID = 16
NGRAPH = 128
ROWBLK = 640


def _elu(x):
    return jnp.where(x > 0, x, jnp.exp(jnp.minimum(x, 0.0)) - 1.0)


def _equi_body(d, blk, x2_ref, x1_ref, wbig_ref, ws_ref, cmain_ref, bias_ref,
               out2_ref, out1_ref):
    inv_d = 1.0 / d
    inv_d2 = inv_d * inv_d
    x4 = x2_ref[...].reshape(blk, d, d, HID)
    x1v = x1_ref[...]
    ii = lax.broadcasted_iota(jnp.int32, (d, d), 0)
    jj = lax.broadcasted_iota(jnp.int32, (d, d), 1)
    mask = (ii == jj).astype(jnp.float32)[None, :, :, None]

    sr = jnp.sum(x4, axis=2) * inv_d
    sc = jnp.sum(x4, axis=1) * inv_d
    dg = jnp.sum(x4 * mask, axis=2)
    sd = jnp.sum(dg, axis=1) * inv_d
    sa = jnp.sum(x4, axis=(1, 2)) * inv_d2
    sx1 = jnp.sum(x1v, axis=1) * inv_d

    rstats = jnp.concatenate([sc, sr, dg, x1v], axis=-1)
    rres = jax.lax.dot_general(rstats, wbig_ref[...],
                               (((2,), (0,)), ((), ())),
                               preferred_element_type=jnp.float32)
    sstats = jnp.concatenate([sd, sa, sx1], axis=-1)
    sres = jnp.dot(sstats, ws_ref[...],
                   preferred_element_type=jnp.float32)

    xcat = jnp.concatenate([x4, jnp.swapaxes(x4, 1, 2)], axis=-1)
    main = jax.lax.dot_general(xcat, cmain_ref[...],
                               (((3,), (0,)), ((), ())),
                               preferred_element_type=jnp.float32)

    b = bias_ref[...]
    out2 = (main
            + rres[:, :, None, 0:32]
            + rres[:, None, :, 32:64]
            + sres[:, None, None, 0:32]
            + b[0, None, None, 0:32]
            + mask * (rres[:, :, None, 64:96]
                      + sres[:, None, None, 32:64]
                      + b[0, None, None, 32:64]))
    out2_ref[...] = _elu(out2).reshape(blk, d * d, 32)

    out1 = rres[:, :, 96:128] + sres[:, None, 64:96] + b[0, None, 64:96]
    out1_ref[...] = _elu(out1)


def _run_equi(x2cl, x1cl, wbig, ws, cmain, biases, d, blk):
    S = x2cl.shape[0]
    grid = S // blk
    body = functools.partial(_equi_body, d, blk)
    return pl.pallas_call(
        body,
        grid=(grid,),
        in_specs=[
            pl.BlockSpec((blk, d * d, HID), lambda i: (i, 0, 0)),
            pl.BlockSpec((blk, d, HID), lambda i: (i, 0, 0)),
            pl.BlockSpec((64, 128), lambda i: (0, 0)),
            pl.BlockSpec((48, 96), lambda i: (0, 0)),
            pl.BlockSpec((32, 32), lambda i: (0, 0)),
            pl.BlockSpec((1, 96), lambda i: (0, 0)),
        ],
        out_specs=[
            pl.BlockSpec((blk, d * d, 32), lambda i: (i, 0, 0)),
            pl.BlockSpec((blk, d, 32), lambda i: (i, 0, 0)),
        ],
        out_shape=[
            jax.ShapeDtypeStruct((S, d * d, 32), jnp.float32),
            jax.ShapeDtypeStruct((S, d, 32), jnp.float32),
        ],
    )(x2cl, x1cl, wbig, ws, cmain, biases)


def _segsum_body(x_ref, seg_ref, s1_ref, s2_ref, cnt_ref):
    @pl.when(pl.program_id(0) == 0)
    def _init():
        s1_ref[...] = jnp.zeros_like(s1_ref)
        s2_ref[...] = jnp.zeros_like(s2_ref)
        cnt_ref[...] = jnp.zeros_like(cnt_ref)

    x = x_ref[...]
    seg = seg_ref[...]
    oh = (seg == lax.broadcasted_iota(jnp.int32, (x.shape[0], NGRAPH), 1)
          ).astype(jnp.float32)
    s1_ref[...] += jax.lax.dot_general(oh, x, (((0,), (0,)), ((), ())),
                                       preferred_element_type=jnp.float32)
    s2_ref[...] += jax.lax.dot_general(oh, x * x, (((0,), (0,)), ((), ())),
                                       preferred_element_type=jnp.float32)
    cnt_ref[...] += jnp.sum(oh, axis=0)[None, :]


def _run_segsum(x, seg):
    N = x.shape[0]
    grid = N // ROWBLK
    return pl.pallas_call(
        _segsum_body,
        grid=(grid,),
        in_specs=[
            pl.BlockSpec((ROWBLK, 32), lambda i: (i, 0)),
            pl.BlockSpec((ROWBLK, 1), lambda i: (i, 0)),
        ],
        out_specs=[
            pl.BlockSpec((NGRAPH, 32), lambda i: (0, 0)),
            pl.BlockSpec((NGRAPH, 32), lambda i: (0, 0)),
            pl.BlockSpec((1, NGRAPH), lambda i: (0, 0)),
        ],
        out_shape=[
            jax.ShapeDtypeStruct((NGRAPH, 32), jnp.float32),
            jax.ShapeDtypeStruct((NGRAPH, 32), jnp.float32),
            jax.ShapeDtypeStruct((1, NGRAPH), jnp.float32),
        ],
    )(x, seg)


def _norm_body(x_ref, seg_ref, scale_ref, shift_ref, out_ref):
    x = x_ref[...]
    seg = seg_ref[...]
    oh = (seg == lax.broadcasted_iota(jnp.int32, (x.shape[0], NGRAPH), 1)
          ).astype(jnp.float32)
    sc = jnp.dot(oh, scale_ref[...], preferred_element_type=jnp.float32)
    sh = jnp.dot(oh, shift_ref[...], preferred_element_type=jnp.float32)
    out_ref[...] = x * sc + sh


def _run_norm(x, seg, scale_t, shift_t):
    N = x.shape[0]
    grid = N // ROWBLK
    return pl.pallas_call(
        _norm_body,
        grid=(grid,),
        in_specs=[
            pl.BlockSpec((ROWBLK, 32), lambda i: (i, 0)),
            pl.BlockSpec((ROWBLK, 1), lambda i: (i, 0)),
            pl.BlockSpec((NGRAPH, 32), lambda i: (0, 0)),
            pl.BlockSpec((NGRAPH, 32), lambda i: (0, 0)),
        ],
        out_specs=pl.BlockSpec((ROWBLK, 32), lambda i: (i, 0)),
        out_shape=jax.ShapeDtypeStruct((N, 32), jnp.float32),
    )(x, seg, scale_t, shift_t)


def _graph_norm(x, seg, p):
    s1, s2, cnt = _run_segsum(x, seg)
    cnt = jnp.maximum(cnt.reshape(NGRAPH, 1), 1.0)
    mu = s1 / cnt
    e2 = s2 / cnt
    a = p['mean_scale'][None, :]
    w = p['weight'][None, :]
    b = p['bias'][None, :]
    var = e2 - (2.0 * a - a * a) * mu * mu
    std = jnp.sqrt(var + 1e-5)
    scale_t = w / std
    shift_t = b - w * a * mu / std
    return _run_norm(x, seg, scale_t, shift_t)


def _build_weights(pd):
    c2 = pd['p2p2']['coeffs']
    c12 = pd['p1p2']['coeffs']
    c21 = pd['p2p1']['coeffs']
    c11 = pd['p1p1']['coeffs']
    Z = jnp.zeros((HID, HID), jnp.float32)

    def lo(m):
        return jnp.concatenate([m, Z], axis=1)

    def hi(m):
        return jnp.concatenate([Z, m], axis=1)

    wrow = jnp.concatenate([lo(c2[:, :, 5]), lo(c2[:, :, 6]),
                            lo(c2[:, :, 11]), hi(c12[:, :, 1])], axis=0)
    wcol = jnp.concatenate([lo(c2[:, :, 7]), lo(c2[:, :, 8]),
                            lo(c2[:, :, 12]), hi(c12[:, :, 2])], axis=0)
    wdiag = jnp.concatenate([lo(c2[:, :, 3]), lo(c2[:, :, 2]),
                             lo(c2[:, :, 0]), hi(c12[:, :, 0])], axis=0)
    wnode = jnp.concatenate([lo(c21[:, :, 3]), lo(c21[:, :, 2]),
                             lo(c21[:, :, 0]), hi(c11[:, :, 0])], axis=0)
    wbig = jnp.concatenate([wrow, wcol, wdiag, wnode], axis=1)

    wconst = jnp.concatenate([lo(c2[:, :, 13]), lo(c2[:, :, 14]),
                              hi(c12[:, :, 4])], axis=0)
    wdiagc = jnp.concatenate([lo(c2[:, :, 1]), lo(c2[:, :, 4]),
                              hi(c12[:, :, 3])], axis=0)
    wnconst = jnp.concatenate([lo(c21[:, :, 1]), lo(c21[:, :, 4]),
                               hi(c11[:, :, 1])], axis=0)
    ws = jnp.concatenate([wconst, wdiagc, wnconst], axis=1)

    cmain = jnp.concatenate([lo(c2[:, :, 9]), lo(c2[:, :, 10])], axis=0)

    bias32 = jnp.concatenate([pd['p2p2']['bias'], pd['p1p2']['bias']])
    dbias32 = jnp.concatenate([pd['p2p2']['diag_bias'], pd['p1p2']['diag_bias']])
    bias132 = jnp.concatenate([pd['p2p1']['bias'], pd['p1p1']['bias']])
    biases = jnp.concatenate([bias32, dbias32, bias132]).reshape(1, 96)
    return wbig, ws, cmain, biases


def kernel(T2_d8, T1_d8, T2_d16, T1_d16, batch, params):
    t2_parts, t1_parts = [], []
    for d, x2, x1, blk in ((8, T2_d8, T1_d8, 16), (16, T2_d16, T1_d16, 8)):
        S = x2.shape[0]
        x2cl = jnp.transpose(x2, (0, 2, 3, 1)).reshape(S, d * d, HID)
        x1cl = jnp.transpose(x1, (0, 2, 1))
        wbig, ws, cmain, biases = _build_weights(params[str(d)])
        o2, o1 = _run_equi(x2cl, x1cl, wbig, ws, cmain, biases, d, blk)
        t2_parts.append(o2.reshape(S * d * d, 32))
        t1_parts.append(o1.reshape(S * d, 32))

    T2 = jnp.concatenate(t2_parts, axis=0)
    T1 = jnp.concatenate(t1_parts, axis=0)

    n8 = T2_d8.shape[0] * 8
    batch = batch.astype(jnp.int32)
    seg1 = batch.reshape(-1, 1)
    seg2 = jnp.concatenate([jnp.repeat(batch[:n8], 8),
                            jnp.repeat(batch[n8:], 16)]).reshape(-1, 1)

    T2 = _graph_norm(T2, seg2, params['gnp2'])
    T1 = _graph_norm(T1, seg1, params['gnp1'])
    return (T2, T1)


# speedup vs baseline: 6.2218x; 1.3032x over previous
import jax
import jax.numpy as jnp
from jax import lax
from jax.experimental import pallas as pl
import functools

HID = 16
NGRAPH = 128
ROWBLK = 640


def _elu(x):
    return jnp.where(x > 0, x, jnp.exp(jnp.minimum(x, 0.0)) - 1.0)


def _onehot(seg, rows):
    return (seg == lax.broadcasted_iota(jnp.int32, (rows, NGRAPH), 1)
            ).astype(jnp.float32)


def _segdot(oh, y):
    return jax.lax.dot_general(oh, y, (((0,), (0,)), ((), ())),
                               preferred_element_type=jnp.float32)


def _equi_body(d, blk, x2_ref, x1_ref, wbig_ref, ws_ref, cmain_ref, bias_ref,
               sege_ref, segn_ref, bufe_ref, bufn_ref,
               out2_ref, out1_ref, s1e_ref, s2e_ref, ce_ref,
               s1n_ref, s2n_ref, cn_ref):
    del bufe_ref, bufn_ref
    inv_d = 1.0 / d
    inv_d2 = inv_d * inv_d
    rows_e = blk * d * d
    rows_n = blk * d
    x4 = x2_ref[...].reshape(blk, d, d, HID)
    x1v = x1_ref[...]
    ii = lax.broadcasted_iota(jnp.int32, (d, d), 0)
    jj = lax.broadcasted_iota(jnp.int32, (d, d), 1)
    mask = (ii == jj).astype(jnp.float32)[None, :, :, None]

    sr = jnp.sum(x4, axis=2) * inv_d
    sc = jnp.sum(x4, axis=1) * inv_d
    dg = jnp.sum(x4 * mask, axis=2)
    sd = jnp.sum(dg, axis=1) * inv_d
    sa = jnp.sum(x4, axis=(1, 2)) * inv_d2
    sx1 = jnp.sum(x1v, axis=1) * inv_d

    rstats = jnp.concatenate([sc, sr, dg, x1v], axis=-1)
    rres = jax.lax.dot_general(rstats, wbig_ref[...],
                               (((2,), (0,)), ((), ())),
                               preferred_element_type=jnp.float32)
    sstats = jnp.concatenate([sd, sa, sx1], axis=-1)
    sres = jnp.dot(sstats, ws_ref[...],
                   preferred_element_type=jnp.float32)

    xcat = jnp.concatenate([x4, jnp.swapaxes(x4, 1, 2)], axis=-1)
    main = jax.lax.dot_general(xcat, cmain_ref[...],
                               (((3,), (0,)), ((), ())),
                               preferred_element_type=jnp.float32)

    b = bias_ref[...]
    out2 = (main
            + rres[:, :, None, 0:32]
            + rres[:, None, :, 32:64]
            + sres[:, None, None, 0:32]
            + b[0, None, None, 0:32]
            + mask * (rres[:, :, None, 64:96]
                      + sres[:, None, None, 32:64]
                      + b[0, None, None, 32:64]))
    y2 = _elu(out2).reshape(rows_e, 32)
    out2_ref[...] = y2

    out1 = rres[:, :, 96:128] + sres[:, None, 64:96] + b[0, None, 64:96]
    y1 = _elu(out1).reshape(rows_n, 32)
    out1_ref[...] = y1

    @pl.when(pl.program_id(0) == 0)
    def _init():
        s1e_ref[...] = jnp.zeros_like(s1e_ref)
        s2e_ref[...] = jnp.zeros_like(s2e_ref)
        ce_ref[...] = jnp.zeros_like(ce_ref)
        s1n_ref[...] = jnp.zeros_like(s1n_ref)
        s2n_ref[...] = jnp.zeros_like(s2n_ref)
        cn_ref[...] = jnp.zeros_like(cn_ref)

    oh_e = _onehot(sege_ref[...], rows_e)
    s1e_ref[...] += _segdot(oh_e, y2)
    s2e_ref[...] += _segdot(oh_e, y2 * y2)
    ce_ref[...] += jnp.sum(oh_e, axis=0)[None, :]
    oh_n = _onehot(segn_ref[...], rows_n)
    s1n_ref[...] += _segdot(oh_n, y1)
    s2n_ref[...] += _segdot(oh_n, y1 * y1)
    cn_ref[...] += jnp.sum(oh_n, axis=0)[None, :]


def _acc_specs():
    specs = [
        pl.BlockSpec((NGRAPH, 32), lambda i: (0, 0)),
        pl.BlockSpec((NGRAPH, 32), lambda i: (0, 0)),
        pl.BlockSpec((1, NGRAPH), lambda i: (0, 0)),
        pl.BlockSpec((NGRAPH, 32), lambda i: (0, 0)),
        pl.BlockSpec((NGRAPH, 32), lambda i: (0, 0)),
        pl.BlockSpec((1, NGRAPH), lambda i: (0, 0)),
    ]
    shapes = [
        jax.ShapeDtypeStruct((NGRAPH, 32), jnp.float32),
        jax.ShapeDtypeStruct((NGRAPH, 32), jnp.float32),
        jax.ShapeDtypeStruct((1, NGRAPH), jnp.float32),
        jax.ShapeDtypeStruct((NGRAPH, 32), jnp.float32),
        jax.ShapeDtypeStruct((NGRAPH, 32), jnp.float32),
        jax.ShapeDtypeStruct((1, NGRAPH), jnp.float32),
    ]
    return specs, shapes


def _run_equi(x2cl, x1cl, wbig, ws, cmain, biases, seg_e, seg_n,
              buf_e, buf_n, d, blk, eoff, noff, ne_total, nn_total, alias):
    S = x2cl.shape[0]
    grid = S // blk
    rows_e = blk * d * d
    rows_n = blk * d
    body = functools.partial(_equi_body, d, blk)
    acc_specs, acc_shapes = _acc_specs()
    in_specs = [
        pl.BlockSpec((blk, d * d, HID), lambda i: (i, 0, 0)),
        pl.BlockSpec((blk, d, HID), lambda i: (i, 0, 0)),
        pl.BlockSpec((64, 128), lambda i: (0, 0)),
        pl.BlockSpec((48, 96), lambda i: (0, 0)),
        pl.BlockSpec((32, 32), lambda i: (0, 0)),
        pl.BlockSpec((1, 96), lambda i: (0, 0)),
        pl.BlockSpec((rows_e, 1), lambda i: (eoff + i, 0)),
        pl.BlockSpec((rows_n, 1), lambda i: (noff + i, 0)),
        pl.BlockSpec((8, 32), lambda i: (0, 0)),
        pl.BlockSpec((8, 32), lambda i: (0, 0)),
    ]
    out_specs = [
        pl.BlockSpec((rows_e, 32), lambda i: (eoff + i, 0)),
        pl.BlockSpec((rows_n, 32), lambda i: (noff + i, 0)),
    ] + acc_specs
    out_shape = [
        jax.ShapeDtypeStruct((ne_total, 32), jnp.float32),
        jax.ShapeDtypeStruct((nn_total, 32), jnp.float32),
    ] + acc_shapes
    return pl.pallas_call(
        body,
        grid=(grid,),
        in_specs=in_specs,
        out_specs=out_specs,
        out_shape=out_shape,
        input_output_aliases={8: 0, 9: 1} if alias else {},
    )(x2cl, x1cl, wbig, ws, cmain, biases, seg_e, seg_n, buf_e, buf_n)


def _norm_body(x_ref, seg_ref, scale_ref, shift_ref, out_ref):
    x = x_ref[...]
    oh = _onehot(seg_ref[...], x.shape[0])
    sc = jnp.dot(oh, scale_ref[...], preferred_element_type=jnp.float32)
    sh = jnp.dot(oh, shift_ref[...], preferred_element_type=jnp.float32)
    out_ref[...] = x * sc + sh


def _run_norm(x, seg, scale_t, shift_t):
    N = x.shape[0]
    grid = N // ROWBLK
    return pl.pallas_call(
        _norm_body,
        grid=(grid,),
        in_specs=[
            pl.BlockSpec((ROWBLK, 32), lambda i: (i, 0)),
            pl.BlockSpec((ROWBLK, 1), lambda i: (i, 0)),
            pl.BlockSpec((NGRAPH, 32), lambda i: (0, 0)),
            pl.BlockSpec((NGRAPH, 32), lambda i: (0, 0)),
        ],
        out_specs=pl.BlockSpec((ROWBLK, 32), lambda i: (i, 0)),
        out_shape=jax.ShapeDtypeStruct((N, 32), jnp.float32),
    )(x, seg, scale_t, shift_t)


def _norm_tables(s1, s2, cnt, p):
    cnt = jnp.maximum(cnt.reshape(NGRAPH, 1), 1.0)
    mu = s1 / cnt
    e2 = s2 / cnt
    a = p['mean_scale'][None, :]
    w = p['weight'][None, :]
    b = p['bias'][None, :]
    var = e2 - (2.0 * a - a * a) * mu * mu
    std = jnp.sqrt(var + 1e-5)
    return w / std, b - w * a * mu / std


def _build_weights(pd):
    c2 = pd['p2p2']['coeffs']
    c12 = pd['p1p2']['coeffs']
    c21 = pd['p2p1']['coeffs']
    c11 = pd['p1p1']['coeffs']
    Z = jnp.zeros((HID, HID), jnp.float32)

    def lo(m):
        return jnp.concatenate([m, Z], axis=1)

    def hi(m):
        return jnp.concatenate([Z, m], axis=1)

    wrow = jnp.concatenate([lo(c2[:, :, 5]), lo(c2[:, :, 6]),
                            lo(c2[:, :, 11]), hi(c12[:, :, 1])], axis=0)
    wcol = jnp.concatenate([lo(c2[:, :, 7]), lo(c2[:, :, 8]),
                            lo(c2[:, :, 12]), hi(c12[:, :, 2])], axis=0)
    wdiag = jnp.concatenate([lo(c2[:, :, 3]), lo(c2[:, :, 2]),
                             lo(c2[:, :, 0]), hi(c12[:, :, 0])], axis=0)
    wnode = jnp.concatenate([lo(c21[:, :, 3]), lo(c21[:, :, 2]),
                             lo(c21[:, :, 0]), hi(c11[:, :, 0])], axis=0)
    wbig = jnp.concatenate([wrow, wcol, wdiag, wnode], axis=1)

    wconst = jnp.concatenate([lo(c2[:, :, 13]), lo(c2[:, :, 14]),
                              hi(c12[:, :, 4])], axis=0)
    wdiagc = jnp.concatenate([lo(c2[:, :, 1]), lo(c2[:, :, 4]),
                              hi(c12[:, :, 3])], axis=0)
    wnconst = jnp.concatenate([lo(c21[:, :, 1]), lo(c21[:, :, 4]),
                               hi(c11[:, :, 1])], axis=0)
    ws = jnp.concatenate([wconst, wdiagc, wnconst], axis=1)

    cmain = jnp.concatenate([lo(c2[:, :, 9]), lo(c2[:, :, 10])], axis=0)

    bias32 = jnp.concatenate([pd['p2p2']['bias'], pd['p1p2']['bias']])
    dbias32 = jnp.concatenate([pd['p2p2']['diag_bias'], pd['p1p2']['diag_bias']])
    bias132 = jnp.concatenate([pd['p2p1']['bias'], pd['p1p1']['bias']])
    biases = jnp.concatenate([bias32, dbias32, bias132]).reshape(1, 96)
    return wbig, ws, cmain, biases


def kernel(T2_d8, T1_d8, T2_d16, T1_d16, batch, params):
    S8 = T2_d8.shape[0]
    S16 = T2_d16.shape[0]
    n8 = S8 * 8
    ne8 = S8 * 64
    ne_total = ne8 + S16 * 256
    nn_total = n8 + S16 * 16

    batch = batch.astype(jnp.int32)
    seg1 = batch.reshape(-1, 1)
    seg2 = jnp.concatenate([jnp.repeat(batch[:n8], 8),
                            jnp.repeat(batch[n8:], 16)]).reshape(-1, 1)

    dummy = jnp.zeros((8, 32), jnp.float32)

    w8 = _build_weights(params['8'])
    x2cl8 = jnp.transpose(T2_d8, (0, 2, 3, 1)).reshape(S8, 64, HID)
    x1cl8 = jnp.transpose(T1_d8, (0, 2, 1))
    r8 = _run_equi(x2cl8, x1cl8, *w8, seg2, seg1, dummy, dummy,
                   8, 16, 0, 0, ne_total, nn_total, alias=False)
    buf_e, buf_n = r8[0], r8[1]

    w16 = _build_weights(params['16'])
    x2cl16 = jnp.transpose(T2_d16, (0, 2, 3, 1)).reshape(S16, 256, HID)
    x1cl16 = jnp.transpose(T1_d16, (0, 2, 1))
    blk16 = 4
    eoff16 = ne8 // (blk16 * 256)
    noff16 = n8 // (blk16 * 16)
    r16 = _run_equi(x2cl16, x1cl16, *w16, seg2, seg1, buf_e, buf_n,
                    16, blk16, eoff16, noff16, ne_total, nn_total, alias=True)

    T2buf, T1buf = r16[0], r16[1]
    s1e = r8[2] + r16[2]
    s2e = r8[3] + r16[3]
    ce = r8[4] + r16[4]
    s1n = r8[5] + r16[5]
    s2n = r8[6] + r16[6]
    cn = r8[7] + r16[7]

    sc2, sh2 = _norm_tables(s1e, s2e, ce, params['gnp2'])
    sc1, sh1 = _norm_tables(s1n, s2n, cn, params['gnp1'])
    T2 = _run_norm(T2buf, seg2, sc2, sh2)
    T1 = _run_norm(T1buf, seg1, sc1, sh1)
    return (T2, T1)
